# baseline (speedup 1.0000x reference)
import functools

import jax
import jax.numpy as jnp
from jax import lax
from jax.experimental import pallas as pl
from jax.experimental.pallas import tpu as pltpu
from jax.experimental.pallas import tpu_sc as plsc

NC, NS, L = 2, 16, 16
NW = NC * NS
B = 16384
D = 32
PACK = 4
VD = PACK * D
BPW = B // NW
CHUNK = 256
NCHUNK = BPW // CHUNK
CGROUPS = CHUNK // L

_mesh = plsc.VectorSubcoreMesh(core_axis_name="c", subcore_axis_name="s",
                               num_cores=NC, num_subcores=NS)


@functools.partial(
    pl.kernel,
    out_type=jax.ShapeDtypeStruct((B,), jnp.float32),
    mesh=_mesh,
    scratch_types=[
        pltpu.VMEM((BPW,), jnp.int32),
        pltpu.VMEM((BPW,), jnp.int32),
        pltpu.VMEM((BPW,), jnp.int32),
        pltpu.VMEM((BPW,), jnp.int32),
        pltpu.VMEM((CHUNK, VD), jnp.float32),
        pltpu.VMEM((CHUNK, VD), jnp.float32),
        pltpu.VMEM((BPW,), jnp.float32),
        pltpu.SemaphoreType.DMA,
        pltpu.SemaphoreType.DMA,
    ],
    compiler_params=pltpu.CompilerParams(use_tc_tiling_on_sc=True,
                                         needs_layout_passes=False),
)
def _dot_kernel(user_hbm, game_hbm, utab_hbm, gtab_hbm, out_hbm,
                uidx_v, gidx_v, uvidx, gvidx, urows, grows, outv,
                sem_u, sem_g):
    wid = lax.axis_index("s") * NC + lax.axis_index("c")
    base = wid * BPW

    pltpu.sync_copy(user_hbm.at[pl.ds(base, BPW)], uidx_v)
    pltpu.sync_copy(game_hbm.at[pl.ds(base, BPW)], gidx_v)

    lane = lax.iota(jnp.int32, L)

    def vidx_body(g, carry):
        sl = pl.ds(g * L, L)
        uvidx[sl] = uidx_v[sl] >> 2
        gvidx[sl] = gidx_v[sl] >> 2
        return carry

    lax.fori_loop(0, BPW // L, vidx_body, 0)

    for c in range(NCHUNK):
        cbase = c * CHUNK
        cu = pltpu.async_copy(utab_hbm.at[uvidx.at[pl.ds(cbase, CHUNK)]],
                              urows, sem_u)
        cg = pltpu.async_copy(gtab_hbm.at[gvidx.at[pl.ds(cbase, CHUNK)]],
                              grows, sem_g)
        cu.wait()
        cg.wait()

        def group_body(g, carry):
            rows = g * L + lane
            u16 = uidx_v[pl.ds(cbase + g * L, L)]
            g16 = gidx_v[pl.ds(cbase + g * L, L)]
            ucol0 = (u16 & 3) * D
            gcol0 = (g16 & 3) * D
            acc = jnp.zeros((L,), jnp.float32)
            for d in range(D):
                vu = plsc.load_gather(urows, [rows, ucol0 + d])
                vg = plsc.load_gather(grows, [rows, gcol0 + d])
                acc = acc + vu * vg
            outv[pl.ds(cbase + g * L, L)] = acc
            return carry

        lax.fori_loop(0, CGROUPS, group_body, 0)

    pltpu.sync_copy(outv, out_hbm.at[pl.ds(base, BPW)])


def kernel(user, game, user_table, game_table):
    nu, ng = user_table.shape[0], game_table.shape[0]
    utab_v = user_table.reshape(nu // PACK, VD)
    gtab_v = game_table.reshape(ng // PACK, VD)
    return _dot_kernel(user.astype(jnp.int32), game.astype(jnp.int32),
                       utab_v, gtab_v)
